# Initial kernel scaffold; baseline (speedup 1.0000x reference)
import jax
import jax.numpy as jnp
from jax.experimental import pallas as pl


def kernel(x, edge_index, W1, b1, W2, b2):
    raise NotImplementedError("write your pallas kernel here")


# speedup vs baseline: 12.9586x; 12.9586x over previous
import functools

import jax
import jax.numpy as jnp
from jax import lax
from jax.experimental import pallas as pl
from jax.experimental.pallas import tpu as pltpu
import jax.experimental.pallas.tpu_sc as plsc

N = 10000
E = 320000
D = 128

NC = 2
NS = 16
NW = NC * NS
EPW = E // NW
CK = 80
NCHUNK = EPW // CK
NP = 10240
RPT = NP // NS
DEGW = 16

_mesh = plsc.VectorSubcoreMesh(core_axis_name="c", subcore_axis_name="s")


@functools.partial(
    pl.kernel,
    out_type=jax.ShapeDtypeStruct((NC, NP, DEGW), jnp.float32),
    mesh=_mesh,
    scratch_types=[
        pltpu.VMEM((CK,), jnp.int32),
        pltpu.VMEM((CK, DEGW), jnp.float32),
        pltpu.VMEM_SHARED((NP, DEGW), jnp.float32),
    ],
)
def _sc_degree(dst_hbm, ones_hbm, zeros_hbm, out_hbm, didx, ones_v, acc):
    c = lax.axis_index("c")
    s = lax.axis_index("s")
    wid = c * NS + s
    pltpu.sync_copy(zeros_hbm, acc.at[pl.ds(s * RPT, RPT)])
    pltpu.sync_copy(ones_hbm, ones_v)
    plsc.subcore_barrier()
    base = wid * EPW

    def body(i, carry):
        pltpu.sync_copy(dst_hbm.at[pl.ds(base + i * CK, CK)], didx)
        pltpu.sync_copy(ones_v, acc.at[didx], add=True)
        return carry

    lax.fori_loop(0, NCHUNK, body, 0)
    plsc.subcore_barrier()
    pltpu.sync_copy(
        acc.at[pl.ds(s * RPT, RPT)], out_hbm.at[c, pl.ds(s * RPT, RPT)]
    )


@functools.partial(
    pl.kernel,
    out_type=jax.ShapeDtypeStruct((NC, NP, D), jnp.float32),
    mesh=_mesh,
    scratch_types=[
        pltpu.VMEM((CK,), jnp.int32),
        pltpu.VMEM((CK,), jnp.int32),
        pltpu.VMEM((CK, D), jnp.float32),
        pltpu.SemaphoreType.DMA,
        pltpu.VMEM_SHARED((NP, D), jnp.float32),
    ],
)
def _sc_scatter(h_hbm, src_hbm, dst_hbm, zeros_hbm, out_hbm,
                sidx, didx, rows, sem, acc):
    c = lax.axis_index("c")
    s = lax.axis_index("s")
    wid = c * NS + s
    pltpu.sync_copy(zeros_hbm, acc.at[pl.ds(s * RPT, RPT)])
    plsc.subcore_barrier()
    base = wid * EPW

    def body(i, carry):
        off = base + i * CK
        pltpu.sync_copy(src_hbm.at[pl.ds(off, CK)], sidx)
        pltpu.sync_copy(dst_hbm.at[pl.ds(off, CK)], didx)
        pltpu.async_copy(h_hbm.at[sidx], rows, sem).wait()
        pltpu.sync_copy(rows, acc.at[didx], add=True)
        return carry

    lax.fori_loop(0, NCHUNK, body, 0)
    plsc.subcore_barrier()
    pltpu.sync_copy(
        acc.at[pl.ds(s * RPT, RPT)], out_hbm.at[c, pl.ds(s * RPT, RPT)]
    )


def _tc_pre_body(x_ref, w_ref, degp_ref, h_ref, dis_ref):
    deg = degp_ref[0, :N, 0:1] + degp_ref[1, :N, 0:1] + 1.0
    dis = lax.rsqrt(deg)
    h = jnp.dot(x_ref[...], w_ref[...], preferred_element_type=jnp.float32)
    h_ref[...] = h * dis
    dis_ref[...] = jnp.broadcast_to(dis, (N, D))


def _tc_mid_body(p_ref, hp_ref, dis_ref, b_ref, w_ref, out_ref):
    s = p_ref[0, :N] + p_ref[1, :N] + hp_ref[...]
    act = jnp.maximum(s * dis_ref[...] + b_ref[...], 0.0)
    h2 = jnp.dot(act, w_ref[...], preferred_element_type=jnp.float32)
    out_ref[...] = h2 * dis_ref[...]


def _tc_post_body(q_ref, hp_ref, dis_ref, b_ref, out_ref):
    s = q_ref[0, :N] + q_ref[1, :N] + hp_ref[...]
    out_ref[...] = s * dis_ref[...] + b_ref[...]


_tc_pre = pl.pallas_call(
    _tc_pre_body,
    out_shape=(
        jax.ShapeDtypeStruct((N, D), jnp.float32),
        jax.ShapeDtypeStruct((N, D), jnp.float32),
    ),
)

_tc_mid = pl.pallas_call(
    _tc_mid_body,
    out_shape=jax.ShapeDtypeStruct((N, D), jnp.float32),
)

_tc_post = pl.pallas_call(
    _tc_post_body,
    out_shape=jax.ShapeDtypeStruct((N, D), jnp.float32),
)


def kernel(x, edge_index, W1, b1, W2, b2):
    src = edge_index[0]
    dst = edge_index[1]
    ones_rows = jnp.ones((CK, DEGW), jnp.float32)
    zeros_deg = jnp.zeros((RPT, DEGW), jnp.float32)
    zeros_rows = jnp.zeros((RPT, D), jnp.float32)

    degp = _sc_degree(dst, ones_rows, zeros_deg)
    h1p, dis = _tc_pre(x, W1, degp)
    p = _sc_scatter(h1p, src, dst, zeros_rows)
    h2p = _tc_mid(p, h1p, dis, b1.reshape(1, D), W2)
    q = _sc_scatter(h2p, src, dst, zeros_rows)
    out = _tc_post(q, h2p, dis, b2.reshape(1, D))
    return out


# speedup vs baseline: 20.3057x; 1.5670x over previous
import functools

import jax
import jax.numpy as jnp
from jax import lax
from jax.experimental import pallas as pl
from jax.experimental.pallas import tpu as pltpu
import jax.experimental.pallas.tpu_sc as plsc

N = 10000
E = 320000
D = 128

NC = 2
NS = 16
NW = NC * NS
EPW = E // NW
CK = 125
NCHUNK = EPW // CK
NBUF = 4
RINGS = NCHUNK // NBUF
NP = 10240
RPT = NP // NS
DEGW = 128

_mesh = plsc.VectorSubcoreMesh(core_axis_name="c", subcore_axis_name="s")


@functools.partial(
    pl.kernel,
    out_type=jax.ShapeDtypeStruct((NC, NP, DEGW), jnp.float32),
    mesh=_mesh,
    scratch_types=[
        pltpu.VMEM((NCHUNK, CK), jnp.int32),
        pltpu.VMEM((CK, DEGW), jnp.float32),
        pltpu.SemaphoreType.DMA,
        pltpu.VMEM_SHARED((NP, DEGW), jnp.float32),
    ],
)
def _sc_degree(dst_hbm, ones_hbm, zeros_hbm, out_hbm, dstv, ones_v, sem, acc):
    c = lax.axis_index("c")
    s = lax.axis_index("s")
    wid = c * NS + s
    pltpu.sync_copy(zeros_hbm, acc.at[pl.ds(s * RPT, RPT)])
    pltpu.sync_copy(ones_hbm, ones_v)
    pltpu.sync_copy(dst_hbm.at[wid], dstv)
    plsc.subcore_barrier()

    def body(j, carry):
        pltpu.sync_copy(ones_v, acc.at[dstv.at[j]], add=True)
        return carry

    lax.fori_loop(0, NCHUNK, body, 0)
    plsc.subcore_barrier()
    pltpu.sync_copy(
        acc.at[pl.ds(s * RPT, RPT)], out_hbm.at[c, pl.ds(s * RPT, RPT)]
    )


@functools.partial(
    pl.kernel,
    out_type=jax.ShapeDtypeStruct((NC, NP, D), jnp.float32),
    mesh=_mesh,
    scratch_types=[
        pltpu.VMEM((NCHUNK, CK), jnp.int32),
        pltpu.VMEM((NCHUNK, CK), jnp.int32),
        pltpu.VMEM((CK, D), jnp.float32),
        pltpu.SemaphoreType.DMA,
        pltpu.VMEM_SHARED((NP, D), jnp.float32),
    ],
)
def _sc_scatter(h_hbm, src_hbm, dst_hbm, zeros_hbm, out_hbm,
                sidx, didx, rows, sem, acc):
    c = lax.axis_index("c")
    s = lax.axis_index("s")
    wid = c * NS + s
    pltpu.sync_copy(zeros_hbm, acc.at[pl.ds(s * RPT, RPT)])
    pltpu.sync_copy(src_hbm.at[wid], sidx)
    pltpu.sync_copy(dst_hbm.at[wid], didx)
    plsc.subcore_barrier()

    def body(i, carry):
        pltpu.async_copy(h_hbm.at[sidx.at[i]], rows, sem).wait()
        pltpu.sync_copy(rows, acc.at[didx.at[i]], add=True)
        return carry

    lax.fori_loop(0, NCHUNK, body, 0)
    plsc.subcore_barrier()
    pltpu.sync_copy(
        acc.at[pl.ds(s * RPT, RPT)], out_hbm.at[c, pl.ds(s * RPT, RPT)]
    )


def _tc_pre_body(x_ref, w_ref, degp_ref, h_ref, dis_ref):
    deg = degp_ref[0, :N, 0:1] + degp_ref[1, :N, 0:1] + 1.0
    dis = lax.rsqrt(deg)
    h = jnp.dot(x_ref[...], w_ref[...], preferred_element_type=jnp.float32)
    h_ref[...] = h * dis
    dis_ref[...] = jnp.broadcast_to(dis, (N, D))


def _tc_mid_body(p_ref, hp_ref, dis_ref, b_ref, w_ref, out_ref):
    s = p_ref[0, :N] + p_ref[1, :N] + hp_ref[...]
    act = jnp.maximum(s * dis_ref[...] + b_ref[...], 0.0)
    h2 = jnp.dot(act, w_ref[...], preferred_element_type=jnp.float32)
    out_ref[...] = h2 * dis_ref[...]


def _tc_post_body(q_ref, hp_ref, dis_ref, b_ref, out_ref):
    s = q_ref[0, :N] + q_ref[1, :N] + hp_ref[...]
    out_ref[...] = s * dis_ref[...] + b_ref[...]


_tc_pre = pl.pallas_call(
    _tc_pre_body,
    out_shape=(
        jax.ShapeDtypeStruct((N, D), jnp.float32),
        jax.ShapeDtypeStruct((N, D), jnp.float32),
    ),
)

_tc_mid = pl.pallas_call(
    _tc_mid_body,
    out_shape=jax.ShapeDtypeStruct((N, D), jnp.float32),
)

_tc_post = pl.pallas_call(
    _tc_post_body,
    out_shape=jax.ShapeDtypeStruct((N, D), jnp.float32),
)


def kernel(x, edge_index, W1, b1, W2, b2):
    src = edge_index[0]
    dst = edge_index[1]
    ones_rows = jnp.ones((CK, DEGW), jnp.float32)
    zeros_deg = jnp.zeros((RPT, DEGW), jnp.float32)
    zeros_rows = jnp.zeros((RPT, D), jnp.float32)

    srcr = src.reshape(NW, NCHUNK, CK)
    dstr = dst.reshape(NW, NCHUNK, CK)
    degp = _sc_degree(dstr, ones_rows, zeros_deg)
    h1p, dis = _tc_pre(x, W1, degp)
    p = _sc_scatter(h1p, srcr, dstr, zeros_rows)
    h2p = _tc_mid(p, h1p, dis, b1.reshape(1, D), W2)
    q = _sc_scatter(h2p, srcr, dstr, zeros_rows)
    out = _tc_post(q, h2p, dis, b2.reshape(1, D))
    return out


# speedup vs baseline: 27.2023x; 1.3396x over previous
import functools

import jax
import jax.numpy as jnp
from jax import lax
from jax.experimental import pallas as pl
from jax.experimental.pallas import tpu as pltpu
import jax.experimental.pallas.tpu_sc as plsc

N = 10000
E = 320000
D = 128

NC = 2
NS = 16
NW = NC * NS
EPW = E // NW
CK = 100
NCHUNK = EPW // CK
NBUF = 2
NHALF = 2
HCHUNK = NCHUNK // NHALF
HRINGS = HCHUNK // NBUF
NP = 10240
RPT = NP // NS
DEGW = 128

_mesh = plsc.VectorSubcoreMesh(core_axis_name="c", subcore_axis_name="s")


@functools.partial(
    pl.kernel,
    out_type=jax.ShapeDtypeStruct((NC, NP, DEGW), jnp.float32),
    mesh=_mesh,
    scratch_types=[
        pltpu.VMEM((NCHUNK, CK), jnp.int32),
        pltpu.VMEM((CK, DEGW), jnp.float32),
        pltpu.SemaphoreType.DMA,
        pltpu.VMEM_SHARED((NP, DEGW), jnp.float32),
    ],
)
def _sc_degree(dst_hbm, ones_hbm, zeros_hbm, out_hbm, dstv, ones_v, sem, acc):
    c = lax.axis_index("c")
    s = lax.axis_index("s")
    wid = c * NS + s
    pltpu.sync_copy(zeros_hbm, acc.at[pl.ds(s * RPT, RPT)])
    pltpu.sync_copy(ones_hbm, ones_v)
    pltpu.sync_copy(dst_hbm.at[wid], dstv)
    plsc.subcore_barrier()

    def body(j, carry):
        pltpu.sync_copy(ones_v, acc.at[dstv.at[j]], add=True)
        return carry

    lax.fori_loop(0, NCHUNK, body, 0)
    plsc.subcore_barrier()
    pltpu.sync_copy(
        acc.at[pl.ds(s * RPT, RPT)], out_hbm.at[c, pl.ds(s * RPT, RPT)]
    )


@functools.partial(
    pl.kernel,
    out_type=jax.ShapeDtypeStruct((NC, NP, D), jnp.float32),
    mesh=_mesh,
    scratch_types=[
        pltpu.VMEM((HCHUNK, CK), jnp.int32),
        pltpu.VMEM((HCHUNK, CK), jnp.int32),
        pltpu.VMEM((NBUF, CK, D), jnp.float32),
        pltpu.SemaphoreType.DMA,
        pltpu.SemaphoreType.DMA,
        pltpu.VMEM_SHARED((NP, D), jnp.float32),
    ],
)
def _sc_scatter(h_hbm, src_hbm, dst_hbm, zeros_hbm, out_hbm,
                sidx, didx, rows, sem0, sem1, acc):
    c = lax.axis_index("c")
    s = lax.axis_index("s")
    wid = c * NS + s
    sems = (sem0, sem1)
    pltpu.sync_copy(zeros_hbm, acc.at[pl.ds(s * RPT, RPT)])
    plsc.subcore_barrier()

    for h in range(NHALF):
        pltpu.sync_copy(src_hbm.at[wid, h], sidx)
        pltpu.sync_copy(dst_hbm.at[wid, h], didx)

        for b in range(NBUF):
            pltpu.async_copy(h_hbm.at[sidx.at[b]], rows.at[b], sems[b])

        def group(r, carry):
            g = r * NBUF
            for b in range(NBUF):
                i = g + b
                pltpu.make_async_copy(
                    h_hbm.at[sidx.at[i]], rows.at[b], sems[b]
                ).wait()
                pltpu.sync_copy(rows.at[b], acc.at[didx.at[i]], add=True)
                pltpu.async_copy(
                    h_hbm.at[sidx.at[i + NBUF]], rows.at[b], sems[b]
                )
            return carry

        lax.fori_loop(0, HRINGS - 1, group, 0)

        gl = (HRINGS - 1) * NBUF
        for b in range(NBUF):
            i = gl + b
            pltpu.make_async_copy(
                h_hbm.at[sidx.at[i]], rows.at[b], sems[b]
            ).wait()
            pltpu.sync_copy(rows.at[b], acc.at[didx.at[i]], add=True)
    plsc.subcore_barrier()
    pltpu.sync_copy(
        acc.at[pl.ds(s * RPT, RPT)], out_hbm.at[c, pl.ds(s * RPT, RPT)]
    )


def _tc_pre_body(x_ref, w_ref, degp_ref, h_ref, dis_ref):
    deg = degp_ref[0, :N, 0:1] + degp_ref[1, :N, 0:1] + 1.0
    dis = lax.rsqrt(deg)
    h = jnp.dot(x_ref[...], w_ref[...], preferred_element_type=jnp.float32)
    h_ref[...] = h * dis
    dis_ref[...] = jnp.broadcast_to(dis, (N, D))


def _tc_mid_body(p_ref, hp_ref, dis_ref, b_ref, w_ref, out_ref):
    s = p_ref[0, :N] + p_ref[1, :N] + hp_ref[...]
    act = jnp.maximum(s * dis_ref[...] + b_ref[...], 0.0)
    h2 = jnp.dot(act, w_ref[...], preferred_element_type=jnp.float32)
    out_ref[...] = h2 * dis_ref[...]


def _tc_post_body(q_ref, hp_ref, dis_ref, b_ref, out_ref):
    s = q_ref[0, :N] + q_ref[1, :N] + hp_ref[...]
    out_ref[...] = s * dis_ref[...] + b_ref[...]


_tc_pre = pl.pallas_call(
    _tc_pre_body,
    out_shape=(
        jax.ShapeDtypeStruct((N, D), jnp.float32),
        jax.ShapeDtypeStruct((N, D), jnp.float32),
    ),
)

_tc_mid = pl.pallas_call(
    _tc_mid_body,
    out_shape=jax.ShapeDtypeStruct((N, D), jnp.float32),
)

_tc_post = pl.pallas_call(
    _tc_post_body,
    out_shape=jax.ShapeDtypeStruct((N, D), jnp.float32),
)


def kernel(x, edge_index, W1, b1, W2, b2):
    src = edge_index[0]
    dst = edge_index[1]
    ones_rows = jnp.ones((CK, DEGW), jnp.float32)
    zeros_deg = jnp.zeros((RPT, DEGW), jnp.float32)
    zeros_rows = jnp.zeros((RPT, D), jnp.float32)

    srcr = src.reshape(NW, NHALF, HCHUNK, CK)
    dstr = dst.reshape(NW, NHALF, HCHUNK, CK)
    degp = _sc_degree(dst.reshape(NW, NCHUNK, CK), ones_rows, zeros_deg)
    h1p, dis = _tc_pre(x, W1, degp)
    p = _sc_scatter(h1p, srcr, dstr, zeros_rows)
    h2p = _tc_mid(p, h1p, dis, b1.reshape(1, D), W2)
    q = _sc_scatter(h2p, srcr, dstr, zeros_rows)
    out = _tc_post(q, h2p, dis, b2.reshape(1, D))
    return out
